# baseline (speedup 1.0000x reference)
import functools

import jax
import jax.numpy as jnp
from jax import lax
from jax.experimental import pallas as pl
from jax.experimental.pallas import tpu as pltpu
from jax.experimental.pallas import tpu_sc as plsc

_B = 16384
_D = 128
_NC = 2
_NS = 16
_NW = _NC * _NS
_BPW = _B // _NW
_CHUNK = 128
_NCHUNK = _BPW // _CHUNK


def _sc_stage(events_features, msg_idx2, idx2, last_update, src, dst):
    mesh = plsc.VectorSubcoreMesh(core_axis_name="c", subcore_axis_name="s")

    @functools.partial(
        pl.kernel,
        out_type=(
            jax.ShapeDtypeStruct((_B, 4 * _D), jnp.float32),
            jax.ShapeDtypeStruct((_B,), jnp.float32),
        ),
        mesh=mesh,
        scratch_types=[
            pltpu.VMEM((_NCHUNK, _CHUNK), jnp.int32),
            pltpu.VMEM((_NCHUNK, _CHUNK), jnp.int32),
            pltpu.VMEM((_BPW, _D), jnp.float32),
            pltpu.VMEM((_BPW,), jnp.float32),
            pltpu.SemaphoreType.DMA,
            pltpu.SemaphoreType.DMA,
            pltpu.SemaphoreType.DMA,
        ],
    )
    def k(ev_hbm, midx_hbm, idx_hbm, lu_hbm, src_hbm, dst_hbm,
          out_hbm, luout_hbm,
          midx_v, idx_v, rows_v, lu_v, sem_e, sem_l, sem_c):
        wid = lax.axis_index("s") * _NC + lax.axis_index("c")
        base = wid * _BPW
        rows = pl.ds(base, _BPW)
        cp_src = pltpu.async_copy(
            src_hbm.at[rows], out_hbm.at[rows, pl.ds(0, _D)], sem_c)
        cp_dst = pltpu.async_copy(
            dst_hbm.at[rows], out_hbm.at[rows, pl.ds(_D, _D)], sem_c)
        pltpu.sync_copy(midx_hbm.at[pl.ds(wid * _NCHUNK, _NCHUNK)], midx_v)
        pltpu.sync_copy(idx_hbm.at[pl.ds(wid * _NCHUNK, _NCHUNK)], idx_v)
        copies = []
        for j in range(_NCHUNK):
            copies.append(pltpu.async_copy(
                ev_hbm.at[midx_v.at[j]],
                rows_v.at[pl.ds(j * _CHUNK, _CHUNK)], sem_e))
            copies.append(pltpu.async_copy(
                lu_hbm.at[idx_v.at[j]],
                lu_v.at[pl.ds(j * _CHUNK, _CHUNK)], sem_l))
        for c in copies:
            c.wait()
        pltpu.sync_copy(rows_v, out_hbm.at[rows, pl.ds(3 * _D, _D)])
        pltpu.sync_copy(lu_v, luout_hbm.at[rows])
        cp_src.wait()
        cp_dst.wait()

    return k(events_features, msg_idx2, idx2, last_update, src, dst)


_BM = 512

_INV_PI = 0.3183098861837907
_PI_HI = 3.140625
_PI_LO = 9.676535897932795e-4


def _fast_cos(x):
    n = jnp.round(x * _INV_PI)
    r = x - n * _PI_HI
    r = r - n * _PI_LO
    u = r * r
    p = 1.0 + u * (-0.5 + u * (1.0 / 24.0 + u * (-1.0 / 720.0
        + u * (1.0 / 40320.0 + u * (-1.0 / 3628800.0)))))
    nh = n * 0.5
    sign = 1.0 - 4.0 * (nh - jnp.floor(nh))
    return sign * p


def _tc_body(ts_ref, lu_ref, w_ref, b_ref, _outal_ref, out_ref):
    dt = ts_ref[...] - lu_ref[...]
    out_ref[...] = _fast_cos(dt * w_ref[...] + b_ref[...])


def _tc_dense(ts2, lu2, w2, b2, out_partial):
    return pl.pallas_call(
        _tc_body,
        out_shape=jax.ShapeDtypeStruct((_B, 4 * _D), jnp.float32),
        grid=(_B // _BM,),
        in_specs=[
            pl.BlockSpec((_BM, 1), lambda i: (i, 0)),
            pl.BlockSpec((_BM, 1), lambda i: (i, 0)),
            pl.BlockSpec((1, _D), lambda i: (0, 0)),
            pl.BlockSpec((1, _D), lambda i: (0, 0)),
            pl.BlockSpec(memory_space=pl.ANY),
        ],
        out_specs=pl.BlockSpec((_BM, _D), lambda i: (i, 2)),
        input_output_aliases={4: 0},
        compiler_params=pltpu.CompilerParams(
            dimension_semantics=("parallel",)),
    )(ts2, lu2, w2, b2, out_partial)


def kernel(src_embeds, dst_embeds, timestamps, last_update, events_features,
           time_w, time_b, idx, msg_indices):
    msg_idx2 = msg_indices.reshape(_B // _CHUNK, _CHUNK)
    idx2 = idx.reshape(_B // _CHUNK, _CHUNK)
    out_partial, lu = _sc_stage(
        events_features, msg_idx2, idx2, last_update, src_embeds, dst_embeds)
    return _tc_dense(
        timestamps.reshape(_B, 1), lu.reshape(_B, 1),
        time_w.reshape(1, _D), time_b.reshape(1, _D),
        out_partial)


# speedup vs baseline: 7.9547x; 7.9547x over previous
import functools

import jax
import jax.numpy as jnp
from jax import lax
from jax.experimental import pallas as pl
from jax.experimental.pallas import tpu as pltpu
from jax.experimental.pallas import tpu_sc as plsc

_B = 16384
_D = 128
_NC = 2
_NS = 16
_NW = _NC * _NS
_BPW = _B // _NW
_CHUNK = 128
_NCHUNK = _BPW // _CHUNK
_NBUF = 4


def _sc_stage(events_features, msg_idx2, idx2, last_update, src, dst):
    mesh = plsc.VectorSubcoreMesh(core_axis_name="c", subcore_axis_name="s")

    @functools.partial(
        pl.kernel,
        out_type=(
            jax.ShapeDtypeStruct((_B, 4 * _D), jnp.float32),
            jax.ShapeDtypeStruct((_B,), jnp.float32),
        ),
        mesh=mesh,
        scratch_types=[
            pltpu.VMEM((_NCHUNK, _CHUNK), jnp.int32),
            pltpu.VMEM((_NCHUNK, _CHUNK), jnp.int32),
            pltpu.VMEM((_NBUF, _CHUNK, _D), jnp.float32),
            pltpu.VMEM((_BPW,), jnp.float32),
            pltpu.SemaphoreType.DMA,
            pltpu.SemaphoreType.DMA,
            pltpu.SemaphoreType.DMA,
            pltpu.SemaphoreType.DMA,
            pltpu.SemaphoreType.DMA,
            pltpu.SemaphoreType.DMA,
        ],
    )
    def k(ev_hbm, midx_hbm, idx_hbm, lu_hbm, src_hbm, dst_hbm,
          out_hbm, luout_hbm,
          midx_v, idx_v, buf_v, lu_v, sem_i, sem_l, s0, s1, s2, s3):
        sems = (s0, s1, s2, s3)
        wid = lax.axis_index("s") * _NC + lax.axis_index("c")
        base = wid * _BPW
        h_mi = pltpu.async_copy(
            midx_hbm.at[pl.ds(wid * _NCHUNK, _NCHUNK)], midx_v, sem_i)
        h_ii = pltpu.async_copy(
            idx_hbm.at[pl.ds(wid * _NCHUNK, _NCHUNK)], idx_v, sem_i)

        units = []
        for j in range(_NCHUNK):
            units += [("s", j), ("d", j), ("e", j)]

        def load_of(u, b):
            kind, j = u
            r = pl.ds(base + j * _CHUNK, _CHUNK)
            if kind == "s":
                return pltpu.async_copy(src_hbm.at[r], buf_v.at[b], sems[b])
            if kind == "d":
                return pltpu.async_copy(dst_hbm.at[r], buf_v.at[b], sems[b])
            return pltpu.async_copy(ev_hbm.at[midx_v.at[j]], buf_v.at[b],
                                    sems[b])

        def store_of(u, b):
            kind, j = u
            r = pl.ds(base + j * _CHUNK, _CHUNK)
            col = {"s": 0, "d": _D, "e": 3 * _D}[kind]
            return pltpu.async_copy(buf_v.at[b], out_hbm.at[r, pl.ds(col, _D)],
                                    sems[b])

        idx_waited = False
        h_ld = [None] * _NBUF
        h_st = [None] * _NBUF
        for u in range(_NBUF):
            if units[u][0] == "e" and not idx_waited:
                h_mi.wait()
                idx_waited = True
            h_ld[u] = load_of(units[u], u)
        h_ii.wait()
        h_lu = [pltpu.async_copy(lu_hbm.at[idx_v.at[j]],
                                 lu_v.at[pl.ds(j * _CHUNK, _CHUNK)], sem_l)
                for j in range(_NCHUNK)]
        for u in range(len(units)):
            b = u % _NBUF
            h_ld[b].wait()
            h_st[b] = store_of(units[u], b)
            nxt = u + _NBUF
            if nxt < len(units):
                if units[nxt][0] == "e" and not idx_waited:
                    h_mi.wait()
                    idx_waited = True
                h_st[b].wait()
                h_ld[b] = load_of(units[nxt], b)
        for b in range(_NBUF):
            if h_st[b] is not None:
                h_st[b].wait()
        for h in h_lu:
            h.wait()
        pltpu.sync_copy(lu_v, luout_hbm.at[pl.ds(base, _BPW)])

    return k(events_features, msg_idx2, idx2, last_update, src, dst)


_BM = 512

_INV_PI = 0.3183098861837907
_PI_HI = 3.140625
_PI_LO = 9.676535897932795e-4


def _fast_cos(x):
    n = jnp.round(x * _INV_PI)
    r = x - n * _PI_HI
    r = r - n * _PI_LO
    u = r * r
    p = 1.0 + u * (-0.5 + u * (1.0 / 24.0 + u * (-1.0 / 720.0
        + u * (1.0 / 40320.0 + u * (-1.0 / 3628800.0)))))
    nh = n * 0.5
    sign = 1.0 - 4.0 * (nh - jnp.floor(nh))
    return sign * p


def _tc_body(ts_ref, lu_ref, w_ref, b_ref, _outal_ref, out_ref):
    dt = ts_ref[...] - lu_ref[...]
    out_ref[...] = _fast_cos(dt * w_ref[...] + b_ref[...])


def _tc_dense(ts2, lu2, w2, b2, out_partial):
    return pl.pallas_call(
        _tc_body,
        out_shape=jax.ShapeDtypeStruct((_B, 4 * _D), jnp.float32),
        grid=(_B // _BM,),
        in_specs=[
            pl.BlockSpec((_BM, 1), lambda i: (i, 0)),
            pl.BlockSpec((_BM, 1), lambda i: (i, 0)),
            pl.BlockSpec((1, _D), lambda i: (0, 0)),
            pl.BlockSpec((1, _D), lambda i: (0, 0)),
            pl.BlockSpec(memory_space=pl.ANY),
        ],
        out_specs=pl.BlockSpec((_BM, _D), lambda i: (i, 2)),
        input_output_aliases={4: 0},
        compiler_params=pltpu.CompilerParams(
            dimension_semantics=("parallel",)),
    )(ts2, lu2, w2, b2, out_partial)


def kernel(src_embeds, dst_embeds, timestamps, last_update, events_features,
           time_w, time_b, idx, msg_indices):
    msg_idx2 = msg_indices.reshape(_B // _CHUNK, _CHUNK)
    idx2 = idx.reshape(_B // _CHUNK, _CHUNK)
    out_partial, lu = _sc_stage(
        events_features, msg_idx2, idx2, last_update, src_embeds, dst_embeds)
    return _tc_dense(
        timestamps.reshape(_B, 1), lu.reshape(_B, 1),
        time_w.reshape(1, _D), time_b.reshape(1, _D),
        out_partial)
